# baseline (speedup 1.0000x reference)
import functools

import jax
import jax.numpy as jnp
from jax import lax
from jax.experimental import pallas as pl
from jax.experimental.pallas import tpu as pltpu
from jax.experimental.pallas import tpu_sc as plsc

_LANES = 16
_CHUNK = 128


@functools.lru_cache(maxsize=None)
def _build(B, D):
    info = plsc.get_sparse_core_info()
    nc, ns = info.num_cores, info.num_subcores
    nw = nc * ns
    assert B % (8 * nw) == 0 and D % _LANES == 0
    b_per_w = B // nw
    chunk = min(_CHUNK, b_per_w)
    n_chunks = b_per_w // chunk
    mesh = plsc.VectorSubcoreMesh(core_axis_name="c", subcore_axis_name="s")

    @functools.partial(
        pl.kernel,
        mesh=mesh,
        out_type=jax.ShapeDtypeStruct((B, D), jnp.float32),
        compiler_params=pltpu.CompilerParams(use_tc_tiling_on_sc=False),
        scratch_types=[
            pltpu.VMEM((b_per_w,), jnp.int32),
            pltpu.VMEM((b_per_w,), jnp.int32),
            pltpu.VMEM((b_per_w, D), jnp.float32),
            pltpu.VMEM((b_per_w, D), jnp.float32),
            pltpu.SemaphoreType.DMA,
        ],
    )
    def gmf(uid_hbm, iid_hbm, ut_hbm, it_hbm, out_hbm,
            uidx_v, iidx_v, urows_v, irows_v, sem):
        wid = lax.axis_index("s") * nc + lax.axis_index("c")
        base = wid * b_per_w
        pltpu.sync_copy(uid_hbm.at[pl.ds(base, b_per_w)], uidx_v)
        pltpu.sync_copy(iid_hbm.at[pl.ds(base, b_per_w)], iidx_v)
        copies = []
        for c in range(n_chunks):
            sl = pl.ds(c * chunk, chunk)
            copies.append(
                pltpu.async_copy(ut_hbm.at[uidx_v.at[sl]], urows_v.at[sl], sem))
            copies.append(
                pltpu.async_copy(it_hbm.at[iidx_v.at[sl]], irows_v.at[sl], sem))
        for cp in copies:
            cp.wait()

        def body(r, carry):
            for j in range(D // _LANES):
                sl2 = pl.ds(j * _LANES, _LANES)
                urows_v[r, sl2] = urows_v[r, sl2] * irows_v[r, sl2]
            return carry

        lax.fori_loop(0, b_per_w, body, 0)
        pltpu.sync_copy(urows_v, out_hbm.at[pl.ds(base, b_per_w)])

    return gmf


def kernel(uid, iid, user_table, item_table):
    B = uid.shape[0]
    D = user_table.shape[1]
    fn = _build(B, D)
    return fn(uid.astype(jnp.int32), iid.astype(jnp.int32),
              user_table, item_table)


# speedup vs baseline: 1.5441x; 1.5441x over previous
import functools

import jax
import jax.numpy as jnp
from jax import lax
from jax.experimental import pallas as pl
from jax.experimental.pallas import tpu as pltpu
from jax.experimental.pallas import tpu_sc as plsc

_LANES = 16


@functools.lru_cache(maxsize=None)
def _build(B, D):
    info = plsc.get_sparse_core_info()
    nc, ns = info.num_cores, info.num_subcores
    nw = nc * ns
    assert B % (8 * nw) == 0 and D % _LANES == 0
    b_per_w = B // nw
    half = b_per_w // 2
    n_groups = half // _LANES
    mesh = plsc.VectorSubcoreMesh(core_axis_name="c", subcore_axis_name="s")

    @functools.partial(
        pl.kernel,
        mesh=mesh,
        out_type=jax.ShapeDtypeStruct((B * D,), jnp.float32),
        compiler_params=pltpu.CompilerParams(use_tc_tiling_on_sc=True),
        scratch_types=[
            pltpu.VMEM((b_per_w,), jnp.int32),
            pltpu.VMEM((b_per_w,), jnp.int32),
            pltpu.VMEM((half, D), jnp.float32),
            pltpu.VMEM((half, D), jnp.float32),
            pltpu.VMEM((b_per_w * D,), jnp.float32),
            pltpu.SemaphoreType.DMA,
            pltpu.SemaphoreType.DMA,
        ],
    )
    def gmf(uid_hbm, iid_hbm, ut_hbm, it_hbm, out_hbm,
            uidx_v, iidx_v, ubuf, ibuf, obuf, usem, isem):
        wid = lax.axis_index("s") * nc + lax.axis_index("c")
        base = wid * b_per_w
        pltpu.sync_copy(uid_hbm.at[pl.ds(base, b_per_w)], uidx_v)
        pltpu.sync_copy(iid_hbm.at[pl.ds(base, b_per_w)], iidx_v)

        def group_wait(sem):
            pltpu.make_async_copy(
                ut_hbm.at[pl.ds(0, _LANES)], ubuf.at[pl.ds(0, _LANES)],
                sem).wait()

        for p in range(2):
            off = p * half

            def issue(g, carry):
                vu = uidx_v[pl.ds(off + g * _LANES, _LANES)]
                vi = iidx_v[pl.ds(off + g * _LANES, _LANES)]
                for l in range(_LANES):
                    j = g * _LANES + l
                    pltpu.async_copy(ut_hbm.at[vu[l]], ubuf.at[j], usem)
                    pltpu.async_copy(it_hbm.at[vi[l]], ibuf.at[j], isem)

                @pl.when(g > 0)
                def _():
                    group_wait(usem)
                    group_wait(isem)

                return carry

            lax.fori_loop(0, n_groups, issue, 0)
            group_wait(usem)
            group_wait(isem)

            def mul(r, carry):
                for k in range(D // _LANES):
                    ksl = pl.ds(k * _LANES, _LANES)
                    osl = pl.ds((off + r) * D + k * _LANES, _LANES)
                    obuf[osl] = ubuf[r, ksl] * ibuf[r, ksl]
                return carry

            lax.fori_loop(0, half, mul, 0)

        pltpu.sync_copy(obuf, out_hbm.at[pl.ds(base * D, b_per_w * D)])

    return gmf


def kernel(uid, iid, user_table, item_table):
    B = uid.shape[0]
    D = user_table.shape[1]
    fn = _build(B, D)
    out = fn(uid.astype(jnp.int32), iid.astype(jnp.int32),
             user_table, item_table)
    return out.reshape(B, D)


# speedup vs baseline: 1.5495x; 1.0035x over previous
import functools

import jax
import jax.numpy as jnp
from jax import lax
from jax.experimental import pallas as pl
from jax.experimental.pallas import tpu as pltpu
from jax.experimental.pallas import tpu_sc as plsc

_LANES = 16


@functools.lru_cache(maxsize=None)
def _build(B, D):
    info = plsc.get_sparse_core_info()
    nc, ns = info.num_cores, info.num_subcores
    nw = nc * ns
    assert B % (8 * nw) == 0 and D % _LANES == 0
    b_per_w = B // nw
    half = b_per_w // 2
    n_groups = half // _LANES
    mesh = plsc.VectorSubcoreMesh(core_axis_name="c", subcore_axis_name="s")

    @functools.partial(
        pl.kernel,
        mesh=mesh,
        out_type=jax.ShapeDtypeStruct((B * D,), jnp.float32),
        compiler_params=pltpu.CompilerParams(use_tc_tiling_on_sc=True),
        scratch_types=[
            pltpu.VMEM((b_per_w,), jnp.int32),
            pltpu.VMEM((b_per_w,), jnp.int32),
            pltpu.VMEM((half, D), jnp.float32),
            pltpu.VMEM((half, D), jnp.float32),
            pltpu.VMEM((b_per_w * D,), jnp.float32),
            pltpu.SemaphoreType.DMA,
            pltpu.SemaphoreType.DMA,
        ],
    )
    def gmf(uid_hbm, iid_hbm, ut_hbm, it_hbm, out_hbm,
            uidx_v, iidx_v, ubuf, ibuf, obuf, usem, isem):
        wid = lax.axis_index("s") * nc + lax.axis_index("c")
        base = wid * b_per_w
        pltpu.sync_copy(uid_hbm.at[pl.ds(base, b_per_w)], uidx_v)
        pltpu.sync_copy(iid_hbm.at[pl.ds(base, b_per_w)], iidx_v)

        def group_wait(sem):
            pltpu.make_async_copy(
                ut_hbm.at[pl.ds(0, _LANES)], ubuf.at[pl.ds(0, _LANES)],
                sem).wait()

        for p in range(2):
            off = p * half

            def issue(g, carry):
                vu = uidx_v[pl.ds(off + g * _LANES, _LANES)]
                vi = iidx_v[pl.ds(off + g * _LANES, _LANES)]
                for l in range(_LANES):
                    j = g * _LANES + l
                    pltpu.async_copy(ut_hbm.at[vu[l]], ubuf.at[j], usem)
                    pltpu.async_copy(it_hbm.at[vi[l]], ibuf.at[j], isem)

                @pl.when(g >= 4)
                def _():
                    group_wait(usem)
                    group_wait(isem)

                return carry

            lax.fori_loop(0, n_groups, issue, 0)
            for _ in range(4):
                group_wait(usem)
                group_wait(isem)

            def mul(r, carry):
                for k in range(D // _LANES):
                    ksl = pl.ds(k * _LANES, _LANES)
                    osl = pl.ds((off + r) * D + k * _LANES, _LANES)
                    obuf[osl] = ubuf[r, ksl] * ibuf[r, ksl]
                return carry

            lax.fori_loop(0, half, mul, 0)

        pltpu.sync_copy(obuf, out_hbm.at[pl.ds(base * D, b_per_w * D)])

    return gmf


def kernel(uid, iid, user_table, item_table):
    B = uid.shape[0]
    D = user_table.shape[1]
    fn = _build(B, D)
    out = fn(uid.astype(jnp.int32), iid.astype(jnp.int32),
             user_table, item_table)
    return out.reshape(B, D)
